# initial kernel scaffold (unmeasured)
import jax
import jax.numpy as jnp
from jax import lax
from jax.experimental import pallas as pl
from jax.experimental.pallas import tpu as pltpu

N_DEV = 32
N_TOK = 1024
D = 256
H = 512
E_LOC = 4
E = N_DEV * E_LOC
CAP = 204


def _ring_allgather(route_idx, expert_W):

    def body(route_ref, w_ref, route_all_ref, w_all_ref,
             local_sem, send_sems, recv_sems, ack_sem):
        my = lax.axis_index("i")
        right = lax.rem(my + 1, N_DEV)
        left = lax.rem(my + N_DEV - 1, N_DEV)

        cp = pltpu.make_async_copy(w_ref, w_all_ref.at[my], local_sem)
        cp.start()
        cp.wait()
        cp = pltpu.make_async_copy(route_ref, route_all_ref.at[my], local_sem)
        cp.start()
        cp.wait()

        def hop(h, carry):
            c = lax.rem(my - h + N_DEV, N_DEV)
            rw = pltpu.make_async_remote_copy(
                src_ref=w_all_ref.at[c],
                dst_ref=w_all_ref.at[c],
                send_sem=send_sems.at[0],
                recv_sem=recv_sems.at[0],
                device_id=(right,),
                device_id_type=pl.DeviceIdType.MESH,
            )
            rr = pltpu.make_async_remote_copy(
                src_ref=route_all_ref.at[c],
                dst_ref=route_all_ref.at[c],
                send_sem=send_sems.at[1],
                recv_sem=recv_sems.at[1],
                device_id=(right,),
                device_id_type=pl.DeviceIdType.MESH,
            )
            rw.start()
            rr.start()
            rw.wait()
            rr.wait()
            pl.semaphore_signal(
                ack_sem, inc=1,
                device_id=(left,), device_id_type=pl.DeviceIdType.MESH,
            )
            pl.semaphore_wait(ack_sem, 1)
            return carry

        lax.fori_loop(0, N_DEV - 1, hop, 0)

    route_all, w_all = pl.pallas_call(
        body,
        out_shape=(
            jax.ShapeDtypeStruct((N_DEV, N_TOK, 1), route_idx.dtype),
            jax.ShapeDtypeStruct((N_DEV, E_LOC, D, H), expert_W.dtype),
        ),
        in_specs=[
            pl.BlockSpec(memory_space=pltpu.ANY),
            pl.BlockSpec(memory_space=pltpu.ANY),
        ],
        out_specs=(
            pl.BlockSpec(memory_space=pltpu.ANY),
            pl.BlockSpec(memory_space=pltpu.ANY),
        ),
        scratch_shapes=[
            pltpu.SemaphoreType.DMA,
            pltpu.SemaphoreType.DMA((2,)),
            pltpu.SemaphoreType.DMA((2,)),
            pltpu.SemaphoreType.REGULAR,
        ],
        compiler_params=pltpu.CompilerParams(collective_id=0),
    )(route_idx, expert_W)
    return route_all, w_all


def kernel(x, router_W, route_idx, expert_W):
    route_all, w_all = _ring_allgather(route_idx, expert_W)

    my = lax.axis_index("i")
    W_all = w_all.reshape(E, D, H)
    route_flat = route_all.reshape(N_DEV * N_TOK).astype(jnp.int32)

    oh = (route_flat[:, None] == jnp.arange(E, dtype=jnp.int32)[None, :])
    oh = oh.astype(jnp.int32)
    pref = jnp.cumsum(oh, axis=0) - oh
    slot_all = jnp.take_along_axis(pref, route_flat[:, None], axis=1)[:, 0]
    my_slot = lax.dynamic_slice(slot_all, (my * N_TOK,), (N_TOK,))

    e_t = route_idx[:, 0].astype(jnp.int32)
    keep = my_slot < CAP
    sl = jnp.clip(my_slot, 0, CAP - 1)

    vals = x * keep[:, None].astype(x.dtype)
    Xd = jnp.zeros((E, CAP, D), x.dtype).at[e_t, sl].add(vals)
    Y = jnp.einsum(
        "ecd,edh->ech", Xd, W_all, preferred_element_type=jnp.float32
    )
    out = Y[e_t, sl] * keep[:, None].astype(x.dtype)
    return out


# baseline (device time: 2810133 ns/iter reference)
import jax
import jax.numpy as jnp
from jax import lax
from jax.experimental import pallas as pl
from jax.experimental.pallas import tpu as pltpu

N_DEV = 32
N_TOK = 1024
D = 256
H = 512
E_LOC = 4
E = N_DEV * E_LOC
CAP = 204


def _ring_allgather(route_idx, expert_W):

    def body(route_ref, w_ref, route_all_ref, w_all_ref,
             local_sem, send_sems, recv_sems, ack_sem):
        my = lax.axis_index("i")
        right = lax.rem(my + 1, N_DEV)
        left = lax.rem(my + N_DEV - 1, N_DEV)

        cp = pltpu.make_async_copy(w_ref, w_all_ref.at[my], local_sem)
        cp.start()
        cp.wait()
        cp = pltpu.make_async_copy(route_ref, route_all_ref.at[my], local_sem)
        cp.start()
        cp.wait()

        def hop(h, carry):
            c = lax.rem(my - h + N_DEV, N_DEV)
            rw = pltpu.make_async_remote_copy(
                src_ref=w_all_ref.at[c],
                dst_ref=w_all_ref.at[c],
                send_sem=send_sems.at[0],
                recv_sem=recv_sems.at[0],
                device_id=(right,),
                device_id_type=pl.DeviceIdType.MESH,
            )
            rr = pltpu.make_async_remote_copy(
                src_ref=route_all_ref.at[c],
                dst_ref=route_all_ref.at[c],
                send_sem=send_sems.at[1],
                recv_sem=recv_sems.at[1],
                device_id=(right,),
                device_id_type=pl.DeviceIdType.MESH,
            )
            rw.start()
            rr.start()
            rw.wait()
            rr.wait()
            pl.semaphore_signal(
                ack_sem, inc=1,
                device_id=(left,), device_id_type=pl.DeviceIdType.MESH,
            )
            pl.semaphore_wait(ack_sem, 1)
            return carry

        lax.fori_loop(0, N_DEV - 1, hop, 0)

    route_all, w_all = pl.pallas_call(
        body,
        out_shape=(
            jax.ShapeDtypeStruct((N_DEV, N_TOK, 1), route_idx.dtype),
            jax.ShapeDtypeStruct((N_DEV, E_LOC, D, H), expert_W.dtype),
        ),
        in_specs=[
            pl.BlockSpec(memory_space=pl.ANY),
            pl.BlockSpec(memory_space=pl.ANY),
        ],
        out_specs=(
            pl.BlockSpec(memory_space=pl.ANY),
            pl.BlockSpec(memory_space=pl.ANY),
        ),
        scratch_shapes=[
            pltpu.SemaphoreType.DMA,
            pltpu.SemaphoreType.DMA((2,)),
            pltpu.SemaphoreType.DMA((2,)),
            pltpu.SemaphoreType.REGULAR,
        ],
    )(route_idx, expert_W)
    return route_all, w_all


def kernel(x, router_W, route_idx, expert_W):
    route_all, w_all = _ring_allgather(route_idx, expert_W)

    my = lax.axis_index("i")
    W_all = w_all.reshape(E, D, H)
    route_flat = route_all.reshape(N_DEV * N_TOK).astype(jnp.int32)

    oh = (route_flat[:, None] == jnp.arange(E, dtype=jnp.int32)[None, :])
    oh = oh.astype(jnp.int32)
    pref = jnp.cumsum(oh, axis=0) - oh
    slot_all = jnp.take_along_axis(pref, route_flat[:, None], axis=1)[:, 0]
    my_slot = lax.dynamic_slice(slot_all, (my * N_TOK,), (N_TOK,))

    e_t = route_idx[:, 0].astype(jnp.int32)
    keep = my_slot < CAP
    sl = jnp.clip(my_slot, 0, CAP - 1)

    vals = x * keep[:, None].astype(x.dtype)
    Xd = jnp.zeros((E, CAP, D), x.dtype).at[e_t, sl].add(vals)
    Y = jnp.einsum(
        "ecd,edh->ech", Xd, W_all, preferred_element_type=jnp.float32
    )
    out = Y[e_t, sl] * keep[:, None].astype(x.dtype)
    return out


# device time: 821244 ns/iter; 3.4218x vs baseline; 3.4218x over previous
import jax
import jax.numpy as jnp
from jax import lax
from jax.experimental import pallas as pl
from jax.experimental.pallas import tpu as pltpu

N_DEV = 32
N_TOK = 1024
D = 256
H = 512
E_LOC = 4
E = N_DEV * E_LOC
CAP = 204


def _hist_allgather(hist):

    def body(hist_ref, out_ref, send_sem, recv_sem, ack_sem):
        my = lax.axis_index("i")
        right = lax.rem(my + 1, N_DEV)
        left = lax.rem(my + N_DEV - 1, N_DEV)

        out_ref[pl.ds(my, 1), :] = hist_ref[...]

        def hop(h, carry):
            c = lax.rem(my - h + N_DEV, N_DEV)
            rdma = pltpu.make_async_remote_copy(
                src_ref=out_ref.at[pl.ds(c, 1)],
                dst_ref=out_ref.at[pl.ds(c, 1)],
                send_sem=send_sem,
                recv_sem=recv_sem,
                device_id=(right,),
                device_id_type=pl.DeviceIdType.MESH,
            )
            rdma.start()
            rdma.wait()
            pl.semaphore_signal(
                ack_sem, inc=1,
                device_id=(left,), device_id_type=pl.DeviceIdType.MESH,
            )
            pl.semaphore_wait(ack_sem, 1)
            return carry

        lax.fori_loop(0, N_DEV - 1, hop, 0)

    return pl.pallas_call(
        body,
        out_shape=jax.ShapeDtypeStruct((N_DEV, E), hist.dtype),
        in_specs=[pl.BlockSpec(memory_space=pltpu.VMEM)],
        out_specs=pl.BlockSpec(memory_space=pltpu.VMEM),
        scratch_shapes=[
            pltpu.SemaphoreType.DMA,
            pltpu.SemaphoreType.DMA,
            pltpu.SemaphoreType.REGULAR,
        ],
    )(hist)


def _moe_ring(x, P, expert_W):

    def body(x_ref, p_ref, w_ref, out_ref, comm_ref,
             send_sems, recv_sems, ack_sem):
        my = lax.axis_index("i")
        right = lax.rem(my + 1, N_DEV)
        left = lax.rem(my + N_DEV - 1, N_DEV)

        comm_ref[0] = w_ref[...]
        xv = x_ref[...]
        pv = p_ref[...]

        def chunk_contrib(c, slot):
            rows = lax.broadcasted_iota(jnp.int32, (E, E_LOC), 0)
            cols = lax.broadcasted_iota(jnp.int32, (E, E_LOC), 1)
            sel = (rows == E_LOC * c + cols).astype(jnp.bfloat16)
            pc = jnp.dot(
                pv, sel, preferred_element_type=jnp.float32
            ).astype(jnp.bfloat16)
            acc = None
            for k in range(E_LOC):
                xm = xv * pc[:, k:k + 1]
                y = jnp.dot(
                    xm, comm_ref[slot, k],
                    preferred_element_type=jnp.float32,
                )
                acc = y if acc is None else acc + y
            return acc

        acc = None
        for h in range(N_DEV - 1):
            s_slot = h % 2
            r_slot = (h + 1) % 2
            c = lax.rem(my - h + N_DEV, N_DEV)
            rdma = pltpu.make_async_remote_copy(
                src_ref=comm_ref.at[s_slot],
                dst_ref=comm_ref.at[r_slot],
                send_sem=send_sems.at[s_slot],
                recv_sem=recv_sems.at[r_slot],
                device_id=(right,),
                device_id_type=pl.DeviceIdType.MESH,
            )
            rdma.start()
            contrib = chunk_contrib(c, s_slot)
            acc = contrib if acc is None else acc + contrib
            rdma.wait()
            pl.semaphore_signal(
                ack_sem, inc=1,
                device_id=(left,), device_id_type=pl.DeviceIdType.MESH,
            )
            pl.semaphore_wait(ack_sem, 1)

        c_last = lax.rem(my + 1, N_DEV)
        acc = acc + chunk_contrib(c_last, (N_DEV - 1) % 2)
        out_ref[...] = acc

    return pl.pallas_call(
        body,
        out_shape=jax.ShapeDtypeStruct((N_TOK, H), jnp.float32),
        in_specs=[
            pl.BlockSpec(memory_space=pltpu.VMEM),
            pl.BlockSpec(memory_space=pltpu.VMEM),
            pl.BlockSpec(memory_space=pltpu.VMEM),
        ],
        out_specs=pl.BlockSpec(memory_space=pltpu.VMEM),
        scratch_shapes=[
            pltpu.VMEM((2, E_LOC, D, H), jnp.bfloat16),
            pltpu.SemaphoreType.DMA((2,)),
            pltpu.SemaphoreType.DMA((2,)),
            pltpu.SemaphoreType.REGULAR,
        ],
    )(
        x.astype(jnp.bfloat16),
        P.astype(jnp.bfloat16),
        expert_W.astype(jnp.bfloat16),
    )


def kernel(x, router_W, route_idx, expert_W):
    e_t = route_idx[:, 0].astype(jnp.int32)
    oh = (e_t[:, None] == jnp.arange(E, dtype=jnp.int32)[None, :])
    oh = oh.astype(jnp.int32)

    loc_excl = jnp.cumsum(oh, axis=0) - oh
    loc = jnp.take_along_axis(loc_excl, e_t[:, None], axis=1)[:, 0]
    hist = oh.sum(axis=0)[None, :]

    hist_all = _hist_allgather(hist)

    my = lax.axis_index("i")
    before = (jnp.arange(N_DEV, dtype=jnp.int32)[:, None] < my)
    base_vec = (hist_all * before.astype(jnp.int32)).sum(axis=0)

    slot = base_vec[e_t] + loc
    keep = slot < CAP
    P = oh.astype(jnp.float32) * keep[:, None].astype(jnp.float32)

    return _moe_ring(x, P, expert_W)
